# baseline (speedup 1.0000x reference)
import functools

import jax
import jax.numpy as jnp
import numpy as np
from jax import lax
from jax.experimental import pallas as pl
from jax.experimental.pallas import tpu as pltpu

_TINY = float(np.finfo(np.float32).tiny)
_BLOCK_ROWS = 8
_CHUNK = 2048
_ARG_CHUNK = 512


def _rotl(v, r):
    return lax.shift_left(v, jnp.uint32(r)) | lax.shift_right_logical(
        v, jnp.uint32(32 - r))


def _threefry_bits(j):
    ks0 = jnp.uint32(0)
    ks1 = jnp.uint32(42)
    ks2 = jnp.uint32(0 ^ 42 ^ 0x1BD11BDA)
    rot0 = (13, 15, 26, 6)
    rot1 = (17, 29, 16, 24)

    def rounds(x0, x1, rots):
        for r in rots:
            x0 = x0 + x1
            x1 = x0 ^ _rotl(x1, r)
        return x0, x1

    x1i = j + ks1
    x0 = x1i
    x1 = x0 ^ _rotl(x1i, 13)
    x0, x1 = rounds(x0, x1, rot0[1:])
    x0, x1 = x0 + ks1, x1 + ks2 + jnp.uint32(1)
    x0, x1 = rounds(x0, x1, rot1)
    x0, x1 = x0 + ks2, x1 + ks0 + jnp.uint32(2)
    x0, x1 = rounds(x0, x1, rot0)
    x0, x1 = x0 + ks0, x1 + ks1 + jnp.uint32(3)
    x0, x1 = rounds(x0, x1, rot1)
    x0, x1 = x0 + ks1, x1 + ks2 + jnp.uint32(4)
    x0, x1 = rounds(x0, x1, rot0)
    x0, x1 = x0 + ks2, x1 + ks0 + jnp.uint32(5)
    return x0 ^ x1


def _gumbel_from_index(j):
    bits = _threefry_bits(j)
    fb = lax.bitcast_convert_type(
        lax.shift_right_logical(bits, jnp.uint32(9)) | jnp.uint32(0x3F800000),
        jnp.float32) - jnp.float32(1.0)
    u = fb + jnp.float32(_TINY)
    return -jnp.log(-jnp.log(u))


def _body(indices, x_ref, o_ref, *, block_rows, n_cols, chunk, arg_chunk):
    pid = indices[0]
    nf = n_cols // chunk
    rem = n_cols - nf * chunk
    na = n_cols // arg_chunk
    rema = n_cols - na * arg_chunk

    shape = (block_rows, chunk)

    mp = x_ref[:, :chunk]
    for k in range(1, nf):
        off = k * chunk
        mp = jnp.maximum(mp, x_ref[:, off:off + chunk])
    m = jnp.max(mp, axis=1, keepdims=True)
    if rem:
        m = jnp.maximum(
            m, jnp.max(x_ref[:, nf * chunk:n_cols], axis=1, keepdims=True))

    sp = jnp.exp(x_ref[:, :chunk] - m)
    for k in range(1, nf):
        off = k * chunk
        sp = sp + jnp.exp(x_ref[:, off:off + chunk] - m)
    s = jnp.sum(sp, axis=1, keepdims=True)
    if rem:
        s = s + jnp.sum(jnp.exp(x_ref[:, nf * chunk:n_cols] - m),
                        axis=1, keepdims=True)
    ashape = (block_rows, arg_chunk)
    base = pid.astype(jnp.uint32) * jnp.uint32(block_rows * n_cols)
    j0 = (base
          + lax.broadcasted_iota(jnp.uint32, ashape, 0) * jnp.uint32(n_cols)
          + lax.broadcasted_iota(jnp.uint32, ashape, 1))
    col_i = lax.broadcasted_iota(jnp.int32, ashape, 1)

    def score_chunk(xc, jc):
        e = jnp.exp(xc - m)
        return jnp.log(e / s) + _gumbel_from_index(jc)

    best = score_chunk(x_ref[:, :arg_chunk], j0)
    bidx = col_i
    for k in range(1, na):
        off = k * arg_chunk
        sc = score_chunk(x_ref[:, off:off + arg_chunk], j0 + jnp.uint32(off))
        bidx = jnp.where(sc > best, col_i + off, bidx)
        best = jnp.maximum(best, sc)
    v = jnp.max(best, axis=1, keepdims=True)
    idx = jnp.min(jnp.where(best == v, bidx, jnp.int32(0x7FFFFFFF)),
                  axis=1, keepdims=True)
    if rema:
        rshape = (block_rows, rema)
        colr_u = lax.broadcasted_iota(jnp.uint32, rshape, 1)
        rowr_u = lax.broadcasted_iota(jnp.uint32, rshape, 0) * jnp.uint32(n_cols)
        off = na * arg_chunk
        sc = score_chunk(x_ref[:, off:n_cols],
                         base + rowr_u + colr_u + jnp.uint32(off))
        vr = jnp.max(sc, axis=1, keepdims=True)
        colr_i = lax.broadcasted_iota(jnp.int32, rshape, 1)
        ir = jnp.min(jnp.where(sc == vr, colr_i + off, jnp.int32(0x7FFFFFFF)),
                     axis=1, keepdims=True)
        idx = jnp.where(vr > v, ir, idx)

    colw_i = lax.broadcasted_iota(jnp.int32, (block_rows, chunk), 1)
    for k in range(nf):
        off = k * chunk
        o_ref[:, off:off + chunk] = (colw_i == idx - off).astype(o_ref.dtype)
    if rem:
        off = nf * chunk
        colr_i = lax.broadcasted_iota(jnp.int32, (block_rows, rem), 1)
        o_ref[:, off:n_cols] = (colr_i == idx - off).astype(o_ref.dtype)


def _outer(x_hbm, o_hbm, *, block_rows, n_rows, n_cols, chunk, arg_chunk):
    pipeline = pltpu.emit_pipeline(
        functools.partial(_body, block_rows=block_rows, n_cols=n_cols,
                          chunk=chunk, arg_chunk=arg_chunk),
        grid=(n_rows // block_rows,),
        in_specs=[pl.BlockSpec((block_rows, n_cols), lambda i: (i, 0))],
        out_specs=[pl.BlockSpec((block_rows, n_cols), lambda i: (i, 0))],
        _explicit_indices=True,
    )
    pipeline(x_hbm, o_hbm)


@jax.jit
def kernel(x):
    n_rows, n_cols = x.shape
    return pl.pallas_call(
        functools.partial(_outer, block_rows=_BLOCK_ROWS, n_rows=n_rows,
                          n_cols=n_cols, chunk=_CHUNK, arg_chunk=_ARG_CHUNK),
        out_shape=jax.ShapeDtypeStruct(x.shape, x.dtype),
        in_specs=[pl.BlockSpec(memory_space=pltpu.MemorySpace.HBM)],
        out_specs=pl.BlockSpec(memory_space=pltpu.MemorySpace.HBM),
    )(x)


# speedup vs baseline: 1.0004x; 1.0004x over previous
import functools

import jax
import jax.numpy as jnp
import numpy as np
from jax import lax
from jax.experimental import pallas as pl
from jax.experimental.pallas import tpu as pltpu

_TINY = float(np.finfo(np.float32).tiny)
_BLOCK_ROWS = 8
_CHUNK = 2048
_ARG_CHUNK = 512


def _rotl(v, r):
    return lax.shift_left(v, jnp.uint32(r)) | lax.shift_right_logical(
        v, jnp.uint32(32 - r))


def _threefry_bits(j):
    ks0 = jnp.uint32(0)
    ks1 = jnp.uint32(42)
    ks2 = jnp.uint32(0 ^ 42 ^ 0x1BD11BDA)
    rot0 = (13, 15, 26, 6)
    rot1 = (17, 29, 16, 24)

    def rounds(x0, x1, rots):
        for r in rots:
            x0 = x0 + x1
            x1 = x0 ^ _rotl(x1, r)
        return x0, x1

    x1i = j + ks1
    x0 = x1i
    x1 = x0 ^ _rotl(x1i, 13)
    x0, x1 = rounds(x0, x1, rot0[1:])
    x0, x1 = x0 + ks1, x1 + ks2 + jnp.uint32(1)
    x0, x1 = rounds(x0, x1, rot1)
    x0, x1 = x0 + ks2, x1 + ks0 + jnp.uint32(2)
    x0, x1 = rounds(x0, x1, rot0)
    x0, x1 = x0 + ks0, x1 + ks1 + jnp.uint32(3)
    x0, x1 = rounds(x0, x1, rot1)
    x0, x1 = x0 + ks1, x1 + ks2 + jnp.uint32(4)
    x0, x1 = rounds(x0, x1, rot0)
    x0, x1 = x0 + ks2, x1 + ks0 + jnp.uint32(5)
    return x0 ^ x1


def _gumbel_from_index(j):
    bits = _threefry_bits(j)
    fb = lax.bitcast_convert_type(
        lax.shift_right_logical(bits, jnp.uint32(9)) | jnp.uint32(0x3F800000),
        jnp.float32) - jnp.float32(1.0)
    u = fb + jnp.float32(_TINY)
    return -jnp.log(-jnp.log(u))


def _body(indices, x_ref, o_ref, *, block_rows, n_cols, chunk, arg_chunk):
    pid = indices[0]
    nf = n_cols // chunk
    rem = n_cols - nf * chunk
    na = n_cols // arg_chunk
    rema = n_cols - na * arg_chunk

    shape = (block_rows, chunk)

    mp = x_ref[:, :chunk]
    for k in range(1, nf):
        off = k * chunk
        mp = jnp.maximum(mp, x_ref[:, off:off + chunk])
    m = jnp.max(mp, axis=1, keepdims=True)
    if rem:
        m = jnp.maximum(
            m, jnp.max(x_ref[:, nf * chunk:n_cols], axis=1, keepdims=True))

    sp = jnp.exp(x_ref[:, :chunk] - m)
    for k in range(1, nf):
        off = k * chunk
        sp = sp + jnp.exp(x_ref[:, off:off + chunk] - m)
    s = jnp.sum(sp, axis=1, keepdims=True)
    if rem:
        s = s + jnp.sum(jnp.exp(x_ref[:, nf * chunk:n_cols] - m),
                        axis=1, keepdims=True)
    ashape = (block_rows, arg_chunk)
    base = pid.astype(jnp.uint32) * jnp.uint32(block_rows * n_cols)
    j0 = (base
          + lax.broadcasted_iota(jnp.uint32, ashape, 0) * jnp.uint32(n_cols)
          + lax.broadcasted_iota(jnp.uint32, ashape, 1))
    col_i = lax.broadcasted_iota(jnp.int32, ashape, 1)

    def score_chunk(xc, jc):
        e = jnp.exp(xc - m)
        return jnp.log(e / s) + _gumbel_from_index(jc)

    best = score_chunk(x_ref[:, :arg_chunk], j0)
    bidx = col_i
    for k in range(1, na):
        off = k * arg_chunk
        sc = score_chunk(x_ref[:, off:off + arg_chunk], j0 + jnp.uint32(off))
        bidx = jnp.where(sc > best, col_i + off, bidx)
        best = jnp.maximum(best, sc)
    v = jnp.max(best, axis=1, keepdims=True)
    idx = jnp.min(jnp.where(best == v, bidx, jnp.int32(0x7FFFFFFF)),
                  axis=1, keepdims=True)
    if rema:
        rshape = (block_rows, rema)
        colr_u = lax.broadcasted_iota(jnp.uint32, rshape, 1)
        rowr_u = lax.broadcasted_iota(jnp.uint32, rshape, 0) * jnp.uint32(n_cols)
        off = na * arg_chunk
        sc = score_chunk(x_ref[:, off:n_cols],
                         base + rowr_u + colr_u + jnp.uint32(off))
        vr = jnp.max(sc, axis=1, keepdims=True)
        colr_i = lax.broadcasted_iota(jnp.int32, rshape, 1)
        ir = jnp.min(jnp.where(sc == vr, colr_i + off, jnp.int32(0x7FFFFFFF)),
                     axis=1, keepdims=True)
        idx = jnp.where(vr > v, ir, idx)

    colw_i = lax.broadcasted_iota(jnp.int32, (block_rows, chunk), 1)
    for k in range(nf):
        off = k * chunk
        o_ref[:, off:off + chunk] = (colw_i == idx - off).astype(o_ref.dtype)
    if rem:
        off = nf * chunk
        colr_i = lax.broadcasted_iota(jnp.int32, (block_rows, rem), 1)
        o_ref[:, off:n_cols] = (colr_i == idx - off).astype(o_ref.dtype)


def _outer(x_hbm, o_hbm, *, block_rows, n_rows, n_cols, chunk, arg_chunk):
    pipeline = pltpu.emit_pipeline(
        functools.partial(_body, block_rows=block_rows, n_cols=n_cols,
                          chunk=chunk, arg_chunk=arg_chunk),
        grid=(n_rows // block_rows,),
        in_specs=[pl.BlockSpec((block_rows, n_cols), lambda i: (i, 0),
                               pipeline_mode=pl.Buffered(buffer_count=3))],
        out_specs=[pl.BlockSpec((block_rows, n_cols), lambda i: (i, 0))],
        _explicit_indices=True,
    )
    pipeline(x_hbm, o_hbm)


@jax.jit
def kernel(x):
    n_rows, n_cols = x.shape
    return pl.pallas_call(
        functools.partial(_outer, block_rows=_BLOCK_ROWS, n_rows=n_rows,
                          n_cols=n_cols, chunk=_CHUNK, arg_chunk=_ARG_CHUNK),
        out_shape=jax.ShapeDtypeStruct(x.shape, x.dtype),
        in_specs=[pl.BlockSpec(memory_space=pltpu.MemorySpace.HBM)],
        out_specs=pl.BlockSpec(memory_space=pltpu.MemorySpace.HBM),
    )(x)


# speedup vs baseline: 1.0550x; 1.0546x over previous
import functools

import jax
import jax.numpy as jnp
import numpy as np
from jax import lax
from jax.experimental import pallas as pl
from jax.experimental.pallas import tpu as pltpu

_TINY = float(np.finfo(np.float32).tiny)
_BLOCK_ROWS = 8
_CHUNK = 2048
_ARG_CHUNK = 512


def _rotl(v, r):
    return lax.shift_left(v, jnp.uint32(r)) | lax.shift_right_logical(
        v, jnp.uint32(32 - r))


def _threefry_bits(j42):
    ks0 = jnp.uint32(0)
    ks1 = jnp.uint32(42)
    ks2 = jnp.uint32(0 ^ 42 ^ 0x1BD11BDA)
    rot0 = (13, 15, 26, 6)
    rot1 = (17, 29, 16, 24)

    def rounds(x0, x1, rots):
        for r in rots:
            x0 = x0 + x1
            x1 = x0 ^ _rotl(x1, r)
        return x0, x1

    x1i = j42
    x0 = x1i
    x1 = x0 ^ _rotl(x1i, 13)
    x0, x1 = rounds(x0, x1, rot0[1:])
    x0, x1 = x0 + ks1, x1 + ks2 + jnp.uint32(1)
    x0, x1 = rounds(x0, x1, rot1)
    x0, x1 = x0 + ks2, x1 + ks0 + jnp.uint32(2)
    x0, x1 = rounds(x0, x1, rot0)
    x0, x1 = x0 + ks0, x1 + ks1 + jnp.uint32(3)
    x0, x1 = rounds(x0, x1, rot1)
    x0, x1 = x0 + ks1, x1 + ks2 + jnp.uint32(4)
    x0, x1 = rounds(x0, x1, rot0)
    x0, x1 = x0 + ks2, x1 + ks0 + jnp.uint32(5)
    return x0 ^ x1


def _body(indices, x_ref, o_ref, *, block_rows, n_cols, chunk, arg_chunk):
    pid = indices[0]
    nf = n_cols // chunk
    rem = n_cols - nf * chunk
    na = n_cols // arg_chunk
    rema = n_cols - na * arg_chunk

    shape = (block_rows, chunk)

    mp = x_ref[:, :chunk]
    for k in range(1, nf):
        off = k * chunk
        mp = jnp.maximum(mp, x_ref[:, off:off + chunk])
    m = jnp.max(mp, axis=1, keepdims=True)
    if rem:
        m = jnp.maximum(
            m, jnp.max(x_ref[:, nf * chunk:n_cols], axis=1, keepdims=True))

    ashape = (block_rows, arg_chunk)
    base = pid.astype(jnp.uint32) * jnp.uint32(block_rows * n_cols)
    j42 = (base + jnp.uint32(42)
           + lax.broadcasted_iota(jnp.uint32, ashape, 0) * jnp.uint32(n_cols)
           + lax.broadcasted_iota(jnp.uint32, ashape, 1))
    col_i = lax.broadcasted_iota(jnp.int32, ashape, 1)

    def score_chunk(xc, jc42):
        bits = _threefry_bits(jc42)
        fb = lax.bitcast_convert_type(
            lax.shift_right_logical(bits, jnp.uint32(9))
            | jnp.uint32(0x3F800000), jnp.float32) - jnp.float32(1.0)
        u = fb + jnp.float32(_TINY)
        w = -jnp.log(u)
        return jnp.exp(xc - m) / w

    best = score_chunk(x_ref[:, :arg_chunk], j42)
    bidx = col_i
    for k in range(1, na):
        off = k * arg_chunk
        sc = score_chunk(x_ref[:, off:off + arg_chunk], j42 + jnp.uint32(off))
        bidx = jnp.where(sc > best, col_i + off, bidx)
        best = jnp.maximum(best, sc)
    v = jnp.max(best, axis=1, keepdims=True)
    idx = jnp.min(jnp.where(best == v, bidx, jnp.int32(0x7FFFFFFF)),
                  axis=1, keepdims=True)
    if rema:
        rshape = (block_rows, rema)
        colr_u = lax.broadcasted_iota(jnp.uint32, rshape, 1)
        rowr_u = lax.broadcasted_iota(jnp.uint32, rshape, 0) * jnp.uint32(n_cols)
        off = na * arg_chunk
        sc = score_chunk(x_ref[:, off:n_cols],
                         base + jnp.uint32(42) + rowr_u + colr_u
                         + jnp.uint32(off))
        vr = jnp.max(sc, axis=1, keepdims=True)
        colr_i = lax.broadcasted_iota(jnp.int32, rshape, 1)
        ir = jnp.min(jnp.where(sc == vr, colr_i + off, jnp.int32(0x7FFFFFFF)),
                     axis=1, keepdims=True)
        idx = jnp.where(vr > v, ir, idx)

    colw_i = lax.broadcasted_iota(jnp.int32, (block_rows, chunk), 1)
    for k in range(nf):
        off = k * chunk
        o_ref[:, off:off + chunk] = (colw_i == idx - off).astype(o_ref.dtype)
    if rem:
        off = nf * chunk
        colr_i = lax.broadcasted_iota(jnp.int32, (block_rows, rem), 1)
        o_ref[:, off:n_cols] = (colr_i == idx - off).astype(o_ref.dtype)


def _outer(x_hbm, o_hbm, *, block_rows, n_rows, n_cols, chunk, arg_chunk):
    pipeline = pltpu.emit_pipeline(
        functools.partial(_body, block_rows=block_rows, n_cols=n_cols,
                          chunk=chunk, arg_chunk=arg_chunk),
        grid=(n_rows // block_rows,),
        in_specs=[pl.BlockSpec((block_rows, n_cols), lambda i: (i, 0),
                               pipeline_mode=pl.Buffered(buffer_count=3))],
        out_specs=[pl.BlockSpec((block_rows, n_cols), lambda i: (i, 0))],
        _explicit_indices=True,
    )
    pipeline(x_hbm, o_hbm)


@jax.jit
def kernel(x):
    n_rows, n_cols = x.shape
    return pl.pallas_call(
        functools.partial(_outer, block_rows=_BLOCK_ROWS, n_rows=n_rows,
                          n_cols=n_cols, chunk=_CHUNK, arg_chunk=_ARG_CHUNK),
        out_shape=jax.ShapeDtypeStruct(x.shape, x.dtype),
        in_specs=[pl.BlockSpec(memory_space=pltpu.MemorySpace.HBM)],
        out_specs=pl.BlockSpec(memory_space=pltpu.MemorySpace.HBM),
    )(x)


# speedup vs baseline: 1.0724x; 1.0165x over previous
import functools

import jax
import jax.numpy as jnp
import numpy as np
from jax import lax
from jax.experimental import pallas as pl
from jax.experimental.pallas import tpu as pltpu

_TINY = float(np.finfo(np.float32).tiny)
_BLOCK_ROWS = 8
_CHUNK = 2048
_ARG_CHUNK = 512


def _rotl(v, r):
    return lax.shift_left(v, jnp.uint32(r)) | lax.shift_right_logical(
        v, jnp.uint32(32 - r))


def _threefry_bits(j42):
    ks0 = jnp.uint32(0)
    ks1 = jnp.uint32(42)
    ks2 = jnp.uint32(0 ^ 42 ^ 0x1BD11BDA)
    rot0 = (13, 15, 26, 6)
    rot1 = (17, 29, 16, 24)

    def rounds(x0, x1, rots):
        for r in rots:
            x0 = x0 + x1
            x1 = x0 ^ _rotl(x1, r)
        return x0, x1

    x1i = j42
    x0 = x1i
    x1 = x0 ^ _rotl(x1i, 13)
    x0, x1 = rounds(x0, x1, rot0[1:])
    x0, x1 = x0 + ks1, x1 + ks2 + jnp.uint32(1)
    x0, x1 = rounds(x0, x1, rot1)
    x0, x1 = x0 + ks2, x1 + ks0 + jnp.uint32(2)
    x0, x1 = rounds(x0, x1, rot0)
    x0, x1 = x0 + ks0, x1 + ks1 + jnp.uint32(3)
    x0, x1 = rounds(x0, x1, rot1)
    x0, x1 = x0 + ks1, x1 + ks2 + jnp.uint32(4)
    x0, x1 = rounds(x0, x1, rot0)
    x0, x1 = x0 + ks2, x1 + ks0 + jnp.uint32(5)
    return x0 ^ x1


def _body(indices, x_ref, o_ref, *, block_rows, n_cols, chunk, arg_chunk):
    pid = indices[0]
    nf = n_cols // chunk
    rem = n_cols - nf * chunk
    na = n_cols // arg_chunk
    rema = n_cols - na * arg_chunk

    ashape = (block_rows, arg_chunk)
    base = pid.astype(jnp.uint32) * jnp.uint32(block_rows * n_cols)
    j42 = (base + jnp.uint32(42)
           + lax.broadcasted_iota(jnp.uint32, ashape, 0) * jnp.uint32(n_cols)
           + lax.broadcasted_iota(jnp.uint32, ashape, 1))
    col_i = lax.broadcasted_iota(jnp.int32, ashape, 1)

    def score_chunk(xc, jc42):
        bits = _threefry_bits(jc42)
        fb = lax.bitcast_convert_type(
            lax.shift_right_logical(bits, jnp.uint32(9))
            | jnp.uint32(0x3F800000), jnp.float32) - jnp.float32(1.0)
        u = fb + jnp.float32(_TINY)
        return xc - jnp.log(-jnp.log(u))

    best = score_chunk(x_ref[:, :arg_chunk], j42)
    bidx = col_i
    for k in range(1, na):
        off = k * arg_chunk
        sc = score_chunk(x_ref[:, off:off + arg_chunk], j42 + jnp.uint32(off))
        bidx = jnp.where(sc > best, col_i + off, bidx)
        best = jnp.maximum(best, sc)
    v = jnp.max(best, axis=1, keepdims=True)
    idx = jnp.min(jnp.where(best == v, bidx, jnp.int32(0x7FFFFFFF)),
                  axis=1, keepdims=True)
    if rema:
        rshape = (block_rows, rema)
        colr_u = lax.broadcasted_iota(jnp.uint32, rshape, 1)
        rowr_u = lax.broadcasted_iota(jnp.uint32, rshape, 0) * jnp.uint32(n_cols)
        off = na * arg_chunk
        sc = score_chunk(x_ref[:, off:n_cols],
                         base + jnp.uint32(42) + rowr_u + colr_u
                         + jnp.uint32(off))
        vr = jnp.max(sc, axis=1, keepdims=True)
        colr_i = lax.broadcasted_iota(jnp.int32, rshape, 1)
        ir = jnp.min(jnp.where(sc == vr, colr_i + off, jnp.int32(0x7FFFFFFF)),
                     axis=1, keepdims=True)
        idx = jnp.where(vr > v, ir, idx)

    colw_i = lax.broadcasted_iota(jnp.int32, (block_rows, chunk), 1)
    for k in range(nf):
        off = k * chunk
        o_ref[:, off:off + chunk] = (colw_i == idx - off).astype(o_ref.dtype)
    if rem:
        off = nf * chunk
        colr_i = lax.broadcasted_iota(jnp.int32, (block_rows, rem), 1)
        o_ref[:, off:n_cols] = (colr_i == idx - off).astype(o_ref.dtype)


def _outer(x_hbm, o_hbm, *, block_rows, n_rows, n_cols, chunk, arg_chunk):
    pipeline = pltpu.emit_pipeline(
        functools.partial(_body, block_rows=block_rows, n_cols=n_cols,
                          chunk=chunk, arg_chunk=arg_chunk),
        grid=(n_rows // block_rows,),
        in_specs=[pl.BlockSpec((block_rows, n_cols), lambda i: (i, 0),
                               pipeline_mode=pl.Buffered(buffer_count=3))],
        out_specs=[pl.BlockSpec((block_rows, n_cols), lambda i: (i, 0))],
        _explicit_indices=True,
    )
    pipeline(x_hbm, o_hbm)


@jax.jit
def kernel(x):
    n_rows, n_cols = x.shape
    return pl.pallas_call(
        functools.partial(_outer, block_rows=_BLOCK_ROWS, n_rows=n_rows,
                          n_cols=n_cols, chunk=_CHUNK, arg_chunk=_ARG_CHUNK),
        out_shape=jax.ShapeDtypeStruct(x.shape, x.dtype),
        in_specs=[pl.BlockSpec(memory_space=pltpu.MemorySpace.HBM)],
        out_specs=pl.BlockSpec(memory_space=pltpu.MemorySpace.HBM),
    )(x)
